# baseline (speedup 1.0000x reference)
import jax
import jax.numpy as jnp
from jax import lax
from jax.experimental import pallas as pl
from jax.experimental.pallas import tpu as pltpu

DIM = 512
KQ = 48000
TEMP = 0.007
TOPN = 32
RANK_K = 4
BSRC = 64
BTGT = 256
KB = 3200
NB = KQ // KB
CHUNK = 128
CPB = KB // CHUNK
NCH = KQ // CHUNK
NCH_PAD = 384

_INTERPRET = False


def _sim_body(tcat_ref, q_ref, srcpad_ref, sim_ref, logz_ref, cmax_ref,
              tnorm_ref, m_ref, s_ref):
    j = pl.program_id(0)

    @pl.when(j == 0)
    def _init():
        x = tcat_ref[...]
        n = jnp.sqrt(jnp.sum(x * x, axis=1, keepdims=True)) + 1e-12
        tnorm_ref[...] = x / n
        m_ref[...] = jnp.full_like(m_ref, -jnp.inf)
        s_ref[...] = jnp.zeros_like(s_ref)

    rows = lax.broadcasted_iota(jnp.int32, (KB, 1), 0) + j * KB
    q = jnp.where(rows < BSRC, srcpad_ref[...], q_ref[...])
    qn = q / (jnp.sqrt(jnp.sum(q * q, axis=1, keepdims=True)) + 1e-12)
    sim = lax.dot_general(tnorm_ref[...], qn, (((1,), (1,)), ((), ())),
                          preferred_element_type=jnp.float32,
                          precision=lax.Precision.HIGHEST)
    sim_ref[...] = sim
    s1 = sim[:BTGT]
    cm = jnp.max(s1.reshape(BTGT, CPB, CHUNK), axis=-1)
    cmax_ref[0] = jnp.concatenate(
        [cm, jnp.full((BTGT, 128 - CPB), -jnp.inf, jnp.float32)], axis=1)
    bm = jnp.max(s1, axis=1, keepdims=True)
    m_old = m_ref[:, :1]
    s_old = s_ref[:, :1]
    m_new = jnp.maximum(m_old, bm)
    s_new = (s_old * jnp.exp((m_old - m_new) / TEMP)
             + jnp.sum(jnp.exp((s1 - m_new) / TEMP), axis=1)[:, None])
    m_ref[:, :1] = m_new
    s_ref[:, :1] = s_new

    @pl.when(j == NB - 1)
    def _fin():
        logz = m_ref[:, :1] / TEMP + jnp.log(s_ref[:, :1])
        logz_ref[...] = logz * jnp.ones((1, 128), jnp.float32)


def _sim_stage(tcat, queue, srcpad):
    return pl.pallas_call(
        _sim_body,
        grid=(NB,),
        in_specs=[
            pl.BlockSpec((2 * BTGT, DIM), lambda j: (0, 0)),
            pl.BlockSpec((KB, DIM), lambda j: (j, 0)),
            pl.BlockSpec((KB, DIM), lambda j: (0, 0)),
        ],
        out_specs=[
            pl.BlockSpec((2 * BTGT, KB), lambda j: (0, j)),
            pl.BlockSpec((BTGT, 128), lambda j: (0, 0)),
            pl.BlockSpec((1, BTGT, 128), lambda j: (j, 0, 0)),
        ],
        out_shape=[
            jax.ShapeDtypeStruct((2 * BTGT, KQ), jnp.float32),
            jax.ShapeDtypeStruct((BTGT, 128), jnp.float32),
            jax.ShapeDtypeStruct((NB, BTGT, 128), jnp.float32),
        ],
        scratch_shapes=[
            pltpu.VMEM((2 * BTGT, DIM), jnp.float32),
            pltpu.VMEM((BTGT, 128), jnp.float32),
            pltpu.VMEM((BTGT, 128), jnp.float32),
        ],
        compiler_params=pltpu.CompilerParams(
            dimension_semantics=("arbitrary",),
            vmem_limit_bytes=100 * 1024 * 1024,
        ),
        interpret=_INTERPRET,
    )(tcat, queue, srcpad)


def kernel(features, target_fearures_0, source_labels, target_labels,
           queue, queue_labels):
    tcat = jnp.concatenate([features[BSRC:], target_fearures_0], axis=0)
    srcpad = jnp.zeros((KB, DIM), jnp.float32).at[:BSRC].set(features[:BSRC])
    sim_all, logz, _cmax = _sim_stage(tcat, queue, srcpad)
    sim = sim_all[:BTGT]
    sim0 = sim_all[BTGT:]
    logZ = logz[:, 0]
    top_vals, top_idx = lax.top_k(sim, TOPN)
    cand0 = jnp.take_along_axis(sim0, top_idx, axis=1)
    rank_scores = top_vals + cand0
    _, rk_order = lax.top_k(rank_scores, RANK_K)
    knn_idx = jnp.take_along_axis(top_idx, rk_order, axis=1)
    labels = queue_labels.at[:BSRC].set(source_labels)
    knn_labels = labels[knn_idx]
    pos = jnp.take_along_axis(sim, knn_idx, axis=1) / TEMP - logZ[:, None]
    sim_loss = -jnp.mean(pos)
    pred = knn_labels[:, 0]
    num_correct = jnp.sum((pred == target_labels).astype(jnp.int64))
    return (sim_loss, num_correct)


# speedup vs baseline: 3.7509x; 3.7509x over previous
import functools

import jax
import jax.numpy as jnp
from jax import lax
from jax.experimental import pallas as pl
from jax.experimental.pallas import tpu as pltpu
from jax.experimental.pallas import tpu_sc as plsc

DIM = 512
KQ = 48000
TEMP = 0.007
TOPN = 32
RANK_K = 4
BSRC = 64
BTGT = 256
KB = 3200
NB = KQ // KB
CHUNK = 128
CPB = KB // CHUNK
NCH = KQ // CHUNK
NCHP = NB * 128
NW = 32
RPW = BTGT // NW
NEG = float("-inf")

_INTERPRET = False


def _sim_body(tcat_ref, q_ref, srcpad_ref, sim_ref, logz_ref, cmax_ref,
              tnorm_ref, m_ref, s_ref):
    j = pl.program_id(0)

    @pl.when(j == 0)
    def _init():
        x = tcat_ref[...]
        n = jnp.sqrt(jnp.sum(x * x, axis=1, keepdims=True)) + 1e-12
        tnorm_ref[...] = x / n
        m_ref[...] = jnp.full_like(m_ref, NEG)
        s_ref[...] = jnp.zeros_like(s_ref)

    rows = lax.broadcasted_iota(jnp.int32, (KB, 1), 0) + j * KB
    q = jnp.where(rows < BSRC, srcpad_ref[...], q_ref[...])
    qn = q / (jnp.sqrt(jnp.sum(q * q, axis=1, keepdims=True)) + 1e-12)
    sim = lax.dot_general(tnorm_ref[...], qn, (((1,), (1,)), ((), ())),
                          preferred_element_type=jnp.float32,
                          precision=lax.Precision.HIGHEST)
    sim_ref[...] = sim
    s1 = sim[:BTGT]
    cm = jnp.max(s1.reshape(BTGT, CPB, CHUNK), axis=-1)
    cmax_ref[...] = jnp.concatenate(
        [cm, jnp.full((BTGT, 128 - CPB), NEG, jnp.float32)], axis=1)
    bm = jnp.max(s1, axis=1, keepdims=True)
    m_old = m_ref[:, :1]
    s_old = s_ref[:, :1]
    m_new = jnp.maximum(m_old, bm)
    s_new = (s_old * jnp.exp((m_old - m_new) / TEMP)
             + jnp.sum(jnp.exp((s1 - m_new) / TEMP), axis=1)[:, None])
    m_ref[:, :1] = m_new
    s_ref[:, :1] = s_new

    @pl.when(j == NB - 1)
    def _fin():
        logz = m_ref[:, :1] / TEMP + jnp.log(s_ref[:, :1])
        logz_ref[...] = logz * jnp.ones((1, 128), jnp.float32)


def _sim_stage(tcat, queue, srcpad):
    return pl.pallas_call(
        _sim_body,
        grid=(NB,),
        in_specs=[
            pl.BlockSpec((2 * BTGT, DIM), lambda j: (0, 0)),
            pl.BlockSpec((KB, DIM), lambda j: (j, 0)),
            pl.BlockSpec((KB, DIM), lambda j: (0, 0)),
        ],
        out_specs=[
            pl.BlockSpec((2 * BTGT, KB), lambda j: (0, j)),
            pl.BlockSpec((BTGT, 128), lambda j: (0, 0)),
            pl.BlockSpec((BTGT, 128), lambda j: (0, j)),
        ],
        out_shape=[
            jax.ShapeDtypeStruct((2 * BTGT, KQ), jnp.float32),
            jax.ShapeDtypeStruct((BTGT, 128), jnp.float32),
            jax.ShapeDtypeStruct((BTGT, NCHP), jnp.float32),
        ],
        scratch_shapes=[
            pltpu.VMEM((2 * BTGT, DIM), jnp.float32),
            pltpu.VMEM((BTGT, 128), jnp.float32),
            pltpu.VMEM((BTGT, 128), jnp.float32),
        ],
        compiler_params=pltpu.CompilerParams(
            dimension_semantics=("arbitrary",),
            vmem_limit_bytes=100 * 1024 * 1024,
        ),
        interpret=_INTERPRET,
    )(tcat, queue, srcpad)


def _lane16():
    return lax.broadcasted_iota(jnp.int32, (16,), 0)


def _argmax_lane(vec, m):
    return jnp.min(jnp.where(vec == m, _lane16(), 99))


def _sstore(ref, idx, val):
    plsc.store_scatter(ref, [jnp.full((16,), idx, jnp.int32)],
                       jnp.full((16,), val, ref.dtype), mask=_lane16() == 0)


def _sstore2(ref, i0, i1, val):
    plsc.store_scatter(ref, [jnp.full((16,), i0, jnp.int32),
                             jnp.full((16,), i1, jnp.int32)],
                       jnp.full((16,), val, ref.dtype), mask=_lane16() == 0)


def _sload(ref, idx):
    return plsc.load_gather(ref, [jnp.full((16,), idx, jnp.int32)])[0]


def _sc_body(sim3, cmax, logz, labs3, tlpad, out_loss, out_corr,
             cmaxbuf, pm, topcm, topchunk, topv, topg, slot32, lane32,
             idx1, idx0, g1, g0, logzbuf, tlbuf, chunk1, lane1, labchunks,
             stage_f, stage_i, sem):
    cid = lax.axis_index("c")
    sid = lax.axis_index("s")
    wid = sid * 2 + cid
    r0 = wid * RPW

    pltpu.sync_copy(logz.at[pl.ds(r0, RPW)], logzbuf)
    pltpu.sync_copy(tlpad.at[pl.ds(r0, 16)], tlbuf)

    def row_body(i, loss_acc):
        r = r0 + i
        pltpu.sync_copy(cmax.at[r], cmaxbuf)

        pm[pl.ds(112, 16)] = jnp.full((16,), NEG, jnp.float32)

        def pm_body(jj, c):
            _sstore(pm, jj, jnp.max(cmaxbuf[pl.ds(jj * 16, 16)]))
            return c

        lax.fori_loop(0, NCHP // 16, pm_body, 0)

        def pick_chunk(i2, c):
            best = jnp.float32(NEG)
            bj = jnp.int32(0)
            for jj in range(8):
                v = jnp.max(pm[pl.ds(jj * 16, 16)])
                upd = v > best
                best = jnp.where(upd, v, best)
                bj = jnp.where(upd, jj, bj)
            grp = bj * 16 + _argmax_lane(pm[pl.ds(bj * 16, 16)], best)
            cvec = cmaxbuf[pl.ds(grp * 16, 16)]
            lane = _argmax_lane(cvec, best)
            pos = grp * 16 + lane
            gch = (pos // 128) * CPB + (pos % 128)
            _sstore(topcm, i2, best)
            _sstore(topchunk, i2, gch)
            _sstore(cmaxbuf, pos, jnp.float32(NEG))
            _sstore(pm, grp, jnp.max(cmaxbuf[pl.ds(grp * 16, 16)]))
            return c

        lax.fori_loop(0, TOPN, pick_chunk, 0)

        ch0 = topchunk[pl.ds(0, 16)]
        ch1 = topchunk[pl.ds(16, 16)]
        idx1[pl.ds(0, 16)] = r * NCH + ch0
        idx1[pl.ds(16, 16)] = r * NCH + ch1
        idx0[pl.ds(0, 16)] = (BTGT + r) * NCH + ch0
        idx0[pl.ds(16, 16)] = (BTGT + r) * NCH + ch1
        pltpu.async_copy(sim3.at[idx1], g1, sem).wait()
        pltpu.async_copy(sim3.at[idx0], g0, sem).wait()

        def pick_val(i2, c):
            best = jnp.float32(NEG)
            bk16 = jnp.int32(0)
            for kk in range(2):
                v = jnp.max(topcm[pl.ds(kk * 16, 16)])
                upd = v > best
                best = jnp.where(upd, v, best)
                bk16 = jnp.where(upd, kk, bk16)
            k = bk16 * 16 + _argmax_lane(topcm[pl.ds(bk16 * 16, 16)], best)
            bv = jnp.int32(0)
            bl = jnp.int32(99)
            for vv in range(8):
                lv = _argmax_lane(g1[k, pl.ds(vv * 16, 16)], best)
                hit = (bl == 99) & (lv < 99)
                bv = jnp.where(hit, vv, bv)
                bl = jnp.where(hit, lv, bl)
            lane = bv * 16 + bl
            _sstore(topv, i2, best)
            _sstore(topg, i2, _sload(topchunk, k) * CHUNK + lane)
            _sstore(slot32, i2, k)
            _sstore(lane32, i2, lane)
            _sstore2(g1, k, lane, jnp.float32(NEG))
            m0 = jnp.float32(NEG)
            for vv in range(8):
                m0 = jnp.maximum(m0, jnp.max(g1[k, pl.ds(vv * 16, 16)]))
            _sstore(topcm, k, m0)
            return c

        lax.fori_loop(0, TOPN, pick_val, 0)

        c0_lo = plsc.load_gather(g0, [slot32[pl.ds(0, 16)],
                                      lane32[pl.ds(0, 16)]])
        c0_hi = plsc.load_gather(g0, [slot32[pl.ds(16, 16)],
                                      lane32[pl.ds(16, 16)]])
        r_lo = topv[pl.ds(0, 16)] + c0_lo
        r_hi = topv[pl.ds(16, 16)] + c0_hi
        vsum = jnp.float32(0.0)
        slot_best = jnp.int32(0)
        for t in range(RANK_K):
            m_lo = jnp.max(r_lo)
            m_hi = jnp.max(r_hi)
            m = jnp.maximum(m_lo, m_hi)
            in_lo = m_lo >= m_hi
            l_lo = _argmax_lane(r_lo, m)
            l_hi = _argmax_lane(r_hi, m)
            slot = jnp.where(in_lo, l_lo, 16 + l_hi)
            vsum = vsum + _sload(topv, slot)
            if t == 0:
                slot_best = slot
            r_lo = jnp.where(in_lo & (_lane16() == l_lo), NEG, r_lo)
            r_hi = jnp.where((~in_lo) & (_lane16() == l_hi), NEG, r_hi)
        gbest = _sload(topg, slot_best)
        _sstore(chunk1, i, gbest // CHUNK)
        _sstore(lane1, i, gbest % CHUNK)
        return (loss_acc + vsum * (1.0 / TEMP)
                - RANK_K * logzbuf[i, pl.ds(0, 16)][0])

    loss_acc = lax.fori_loop(0, RPW, row_body, jnp.float32(0.0))

    pltpu.async_copy(labs3.at[chunk1], labchunks, sem).wait()

    def corr_body(i, c):
        lv = plsc.load_gather(lane1, [jnp.full((16,), i, jnp.int32)])
        lab = plsc.load_gather(labchunks,
                               [jnp.full((16,), i, jnp.int32), lv])[0]
        return c + jnp.where(lab == _sload(tlbuf, i), 1, 0)

    corr = lax.fori_loop(0, RPW, corr_body, jnp.int32(0))

    stage_f[...] = jnp.full((16,), loss_acc, jnp.float32)
    stage_i[...] = jnp.full((16,), corr, jnp.int32)
    pltpu.sync_copy(stage_f, out_loss.at[wid])
    pltpu.sync_copy(stage_i, out_corr.at[wid])


def _sparse_stage(sim3, cmax, logz, labs3, tlpad):
    mesh = plsc.VectorSubcoreMesh(core_axis_name="c", subcore_axis_name="s")
    f = functools.partial(
        pl.kernel,
        out_type=[
            jax.ShapeDtypeStruct((NW, 16), jnp.float32),
            jax.ShapeDtypeStruct((NW, 16), jnp.int32),
        ],
        mesh=mesh,
        scratch_types=[
            pltpu.VMEM((NCHP,), jnp.float32),
            pltpu.VMEM((128,), jnp.float32),
            pltpu.VMEM((TOPN,), jnp.float32),
            pltpu.VMEM((TOPN,), jnp.int32),
            pltpu.VMEM((TOPN,), jnp.float32),
            pltpu.VMEM((TOPN,), jnp.int32),
            pltpu.VMEM((TOPN,), jnp.int32),
            pltpu.VMEM((TOPN,), jnp.int32),
            pltpu.VMEM((TOPN,), jnp.int32),
            pltpu.VMEM((TOPN,), jnp.int32),
            pltpu.VMEM((TOPN, CHUNK), jnp.float32),
            pltpu.VMEM((TOPN, CHUNK), jnp.float32),
            pltpu.VMEM((RPW, 128), jnp.float32),
            pltpu.VMEM((16,), jnp.int32),
            pltpu.VMEM((RPW,), jnp.int32),
            pltpu.VMEM((RPW,), jnp.int32),
            pltpu.VMEM((RPW, CHUNK), jnp.int32),
            pltpu.VMEM((16,), jnp.float32),
            pltpu.VMEM((16,), jnp.int32),
            pltpu.SemaphoreType.DMA,
        ],
        compiler_params=pltpu.CompilerParams(needs_layout_passes=False),
        interpret=_INTERPRET,
    )(_sc_body)
    return f(sim3, cmax, logz, labs3, tlpad)


def kernel(features, target_fearures_0, source_labels, target_labels,
           queue, queue_labels):
    tcat = jnp.concatenate([features[BSRC:], target_fearures_0], axis=0)
    srcpad = jnp.zeros((KB, DIM), jnp.float32).at[:BSRC].set(features[:BSRC])
    sim_all, logz, cmax = _sim_stage(tcat, queue, srcpad)
    sim3 = sim_all.reshape(2 * BTGT * NCH, CHUNK)
    labs3 = queue_labels.at[:BSRC].set(source_labels).reshape(NCH, CHUNK)
    tlpad = jnp.pad(target_labels, (0, 16))
    out_loss, out_corr = _sparse_stage(sim3, cmax, logz, labs3, tlpad)
    sim_loss = -jnp.sum(out_loss[:, 0]) / (BTGT * RANK_K)
    num_correct = jnp.sum(out_corr[:, 0]).astype(jnp.int64)
    return (sim_loss, num_correct)
